# Initial kernel scaffold; baseline (speedup 1.0000x reference)
import jax
import jax.numpy as jnp
from jax.experimental import pallas as pl


def _rownorm(v, g):
    n = jnp.maximum(jnp.linalg.norm(v, axis=1, keepdims=True), 1e-12)
    return (v / n) * g[:, None]


def kernel(x, v1, g1, b1, v2, g2, b2, gv, gg, gb):
    b, s, d = x.shape
    xf = x.reshape(-1, d)
    Wg = _rownorm(gv, gg)
    logits = jax.lax.dot_general(xf, Wg, (((1,), (1,)), ((), ())),
                                 precision=jax.lax.Precision.HIGHEST) + gb
    gates = jax.nn.softmax(logits, axis=-1)
    top_values, top_indices = jax.lax.top_k(gates, 2)
    expert_outputs = jnp.zeros_like(xf)
    for i in range(8):
        mask = jnp.any(top_indices == i, axis=1).astype(xf.dtype)[:, None]
        expert_input = xf * mask
        W1 = _rownorm(v1[i], g1[i])
        h = expert_input @ W1.T + b1[i]
        h = h * jax.nn.sigmoid(h)
        W2 = _rownorm(v2[i], g2[i])
        y = h @ W2.T + b2[i]
        expert_outputs = expert_outputs + mask * y
    return expert_outputs.reshape(b, s, d)


# speedup vs baseline: 1.7163x; 1.7163x over previous
import jax
import jax.numpy as jnp
from jax import lax
from jax.experimental import pallas as pl
from jax.experimental.pallas import tpu as pltpu
from jax.experimental.pallas import tpu_sc as plsc

N_TOK = 4096
D = 1024
H = 4096
E = 8
EPAD = 128
NEG = -1e30
TM = 256
BLK = 512
NT = 40
P_ROWS = NT * TM

NC = 2
NS = 16
NW = NC * NS
TPW = N_TOK // NW
CH = 32
NCH = TPW // CH


def _gate_kernel(xf_ref, wg_ref, gb_ref, lt_ref, pos1_ref, pos2_ref, et_ref):
    x = xf_ref[...]
    wg = wg_ref[...]
    logits = jax.lax.dot_general(
        x, wg, (((1,), (1,)), ((), ())),
        preferred_element_type=jnp.float32)
    logits = logits + gb_ref[...]
    lane = jax.lax.broadcasted_iota(jnp.int32, logits.shape, 1)
    l = jnp.where(lane < E, logits, NEG)
    m1 = jnp.max(l, axis=1, keepdims=True)
    i1 = jnp.min(jnp.where(l == m1, lane, EPAD - 1), axis=1, keepdims=True)
    l2 = jnp.where(lane == i1, NEG, l)
    m2 = jnp.max(l2, axis=1, keepdims=True)
    i2 = jnp.min(jnp.where(l2 == m2, lane, EPAD - 1), axis=1, keepdims=True)
    oh1 = lane == i1
    oh2 = lane == i2
    mask = (oh1 | oh2).astype(jnp.float32)
    lt = lt_ref[...]
    run = jnp.zeros((1, EPAD), jnp.float32)
    parts = []
    for bk in range(N_TOK // BLK):
        m_b = mask[bk * BLK:(bk + 1) * BLK, :]
        c_b = jax.lax.dot_general(lt, m_b, (((1,), (0,)), ((), ())),
                                  preferred_element_type=jnp.float32)
        parts.append(c_b + run)
        run = run + c_b[BLK - 1:BLK, :]
    cum = jnp.concatenate(parts, axis=0)
    counts = run.astype(jnp.int32)
    padded = ((counts + (TM - 1)) >> 8) << 8
    lane_r = jax.lax.broadcasted_iota(jnp.int32, (1, EPAD), 1)
    offs = jnp.zeros((1, EPAD), jnp.int32)
    for e in range(E - 1):
        p_e = jnp.sum(jnp.where(lane_r == e, padded, 0))
        offs = offs + jnp.where(lane_r > e, p_e, 0)
    pos = offs.astype(jnp.float32) + cum - 1.0
    pos1_ref[...] = jnp.sum(jnp.where(oh1, pos, 0.0), axis=1,
                            keepdims=True).astype(jnp.int32)
    pos2_ref[...] = jnp.sum(jnp.where(oh2, pos, 0.0), axis=1,
                            keepdims=True).astype(jnp.int32)
    lane_row = jax.lax.broadcasted_iota(jnp.int32, (1, EPAD), 1)
    tl = lane_row * TM
    et = jnp.zeros((1, EPAD), jnp.int32)
    for e in range(E):
        off_e = jnp.sum(jnp.where(lane_row == e, offs, 0))
        et = et + (tl >= off_e).astype(jnp.int32)
    et_ref[...] = et - 1


def _dispatch_kernel(xf_hbm, pos1_hbm, pos2_hbm, xg_hbm,
                     idx1_v, idx2_v, rows_v, sem_in, sem_s1, sem_s2):
    wid = lax.axis_index("s") * NC + lax.axis_index("c")
    base = wid * TPW
    pltpu.sync_copy(pos1_hbm.at[wid], idx1_v)
    pltpu.sync_copy(pos2_hbm.at[wid], idx2_v)
    for j in range(NCH):
        pltpu.async_copy(xf_hbm.at[pl.ds(base + j * CH, CH)], rows_v,
                         sem_in).wait()
        cp1 = pltpu.async_copy(rows_v, xg_hbm.at[idx1_v.at[j]], sem_s1)
        cp2 = pltpu.async_copy(rows_v, xg_hbm.at[idx2_v.at[j]], sem_s2)
        cp1.wait()
        cp2.wait()


def _gemm_kernel(et_ref, xg_ref, v1_ref, v2_ref, g1_ref, b1_ref, g2_ref,
                 b2_ref, o_ref, s1_ref, s2_ref):
    t = pl.program_id(0)
    w1 = v1_ref[0]
    w2 = v2_ref[0]
    first = t == 0
    changed = et_ref[t] != et_ref[jnp.maximum(t - 1, 0)]

    @pl.when(jnp.logical_or(first, changed))
    def _():
        w1f = w1.astype(jnp.float32)
        ss1 = jnp.sum(w1f * w1f, axis=0, keepdims=True)
        s1_ref[...] = g1_ref[0] * jax.lax.rsqrt(jnp.maximum(ss1, 1e-24))
        w2f = w2.astype(jnp.float32)
        ss2 = jnp.sum(w2f * w2f, axis=0, keepdims=True)
        s2_ref[...] = g2_ref[0] * jax.lax.rsqrt(jnp.maximum(ss2, 1e-24))

    x = xg_ref[...].astype(jnp.bfloat16)
    h = jax.lax.dot_general(x, w1, (((1,), (0,)), ((), ())),
                            preferred_element_type=jnp.float32)
    h = h * s1_ref[...] + b1_ref[0]
    h = h / (1.0 + jnp.exp(-h))
    hb = h.astype(jnp.bfloat16)
    y = jax.lax.dot_general(hb, w2, (((1,), (0,)), ((), ())),
                            preferred_element_type=jnp.float32)
    o_ref[...] = y * s2_ref[...] + b2_ref[0]


def _combine_kernel(yg_hbm, pos1_hbm, pos2_hbm, out_hbm,
                    idx1_v, idx2_v, r1_v, r2_v, sem_g1, sem_g2):
    wid = lax.axis_index("s") * NC + lax.axis_index("c")
    base = wid * TPW
    pltpu.sync_copy(pos1_hbm.at[wid], idx1_v)
    pltpu.sync_copy(pos2_hbm.at[wid], idx2_v)
    for j in range(NCH):
        pltpu.async_copy(yg_hbm.at[idx1_v.at[j]], r1_v, sem_g1).wait()
        pltpu.async_copy(yg_hbm.at[idx2_v.at[j]], r2_v, sem_g2).wait()

        def body(c, _):
            for r in range(CH):
                sl = pl.ds(c * 16, 16)
                r1_v[r, sl] = r1_v[r, sl] + r2_v[r, sl]
            return 0

        lax.fori_loop(0, D // 16, body, 0)
        pltpu.sync_copy(r1_v, out_hbm.at[pl.ds(base + j * CH, CH)])


def kernel(x, v1, g1, b1, v2, g2, b2, gv, gg, gb):
    b, s, d = x.shape
    xf = x.reshape(-1, d)
    n = jnp.maximum(jnp.linalg.norm(gv, axis=1, keepdims=True), 1e-12)
    wg = (gv / n) * gg[:, None]
    wg_pad = jnp.zeros((EPAD, D), jnp.float32).at[:E].set(wg)
    gb_pad = jnp.zeros((1, EPAD), jnp.float32).at[0, :E].set(gb)

    lt = jnp.tril(jnp.ones((BLK, BLK), jnp.float32))

    pos1, pos2, et = pl.pallas_call(
        _gate_kernel,
        out_shape=[jax.ShapeDtypeStruct((N_TOK, 1), jnp.int32),
                   jax.ShapeDtypeStruct((N_TOK, 1), jnp.int32),
                   jax.ShapeDtypeStruct((1, EPAD), jnp.int32)],
    )(xf, wg_pad, gb_pad, lt)

    pos1r = pos1.reshape(NW, NCH, CH)
    pos2r = pos2.reshape(NW, NCH, CH)
    et_flat = et.reshape(EPAD)[:NT]

    mesh = plsc.VectorSubcoreMesh(core_axis_name="c", subcore_axis_name="s")
    xg = pl.kernel(
        _dispatch_kernel,
        out_type=jax.ShapeDtypeStruct((P_ROWS, D), jnp.float32),
        mesh=mesh,
        scratch_types=[pltpu.VMEM((NCH, CH), jnp.int32),
                       pltpu.VMEM((NCH, CH), jnp.int32),
                       pltpu.VMEM((CH, D), jnp.float32),
                       pltpu.SemaphoreType.DMA,
                       pltpu.SemaphoreType.DMA,
                       pltpu.SemaphoreType.DMA],
    )(xf, pos1r, pos2r)

    v1t = jnp.swapaxes(v1.astype(jnp.bfloat16), 1, 2)
    v2t = jnp.swapaxes(v2.astype(jnp.bfloat16), 1, 2)

    yg = pl.pallas_call(
        _gemm_kernel,
        grid_spec=pltpu.PrefetchScalarGridSpec(
            num_scalar_prefetch=1,
            grid=(NT,),
            in_specs=[
                pl.BlockSpec((TM, D), lambda t, et_r: (t, 0)),
                pl.BlockSpec((1, D, H), lambda t, et_r: (et_r[t], 0, 0)),
                pl.BlockSpec((1, H, D), lambda t, et_r: (et_r[t], 0, 0)),
                pl.BlockSpec((1, 1, H), lambda t, et_r: (et_r[t], 0, 0)),
                pl.BlockSpec((1, 1, H), lambda t, et_r: (et_r[t], 0, 0)),
                pl.BlockSpec((1, 1, D), lambda t, et_r: (et_r[t], 0, 0)),
                pl.BlockSpec((1, 1, D), lambda t, et_r: (et_r[t], 0, 0)),
            ],
            out_specs=pl.BlockSpec((TM, D), lambda t, et_r: (t, 0)),
            scratch_shapes=[pltpu.VMEM((1, H), jnp.float32),
                            pltpu.VMEM((1, D), jnp.float32)],
        ),
        out_shape=jax.ShapeDtypeStruct((P_ROWS, D), jnp.float32),
        compiler_params=pltpu.CompilerParams(
            vmem_limit_bytes=100 * 1024 * 1024),
    )(et_flat, xg, v1t, v2t, g1.reshape(E, 1, H), b1.reshape(E, 1, H),
      g2.reshape(E, 1, D), b2.reshape(E, 1, D))

    out = pl.kernel(
        _combine_kernel,
        out_type=jax.ShapeDtypeStruct((N_TOK, D), jnp.float32),
        mesh=mesh,
        scratch_types=[pltpu.VMEM((NCH, CH), jnp.int32),
                       pltpu.VMEM((NCH, CH), jnp.int32),
                       pltpu.VMEM((CH, D), jnp.float32),
                       pltpu.VMEM((CH, D), jnp.float32),
                       pltpu.SemaphoreType.DMA,
                       pltpu.SemaphoreType.DMA],
    )(yg, pos1r, pos2r)

    return out.reshape(b, s, d)


# speedup vs baseline: 1.8582x; 1.0826x over previous
import jax
import jax.numpy as jnp
from jax import lax
from jax.experimental import pallas as pl
from jax.experimental.pallas import tpu as pltpu
from jax.experimental.pallas import tpu_sc as plsc

N_TOK = 4096
D = 1024
H = 4096
E = 8
EPAD = 128
NEG = -1e30
TM = 512
TM_LOG2 = 9
BLK = 512
NT = 23
P_ROWS = NT * TM

NC = 2
NS = 16
NW = NC * NS
TPW = N_TOK // NW
CH = 32
NCH = TPW // CH


def _gate_kernel(xf_ref, wg_ref, gb_ref, lt_ref, pos1_ref, pos2_ref, et_ref):
    x = xf_ref[...]
    wg = wg_ref[...]
    logits = jax.lax.dot_general(
        x, wg, (((1,), (1,)), ((), ())),
        preferred_element_type=jnp.float32)
    logits = logits + gb_ref[...]
    lane = jax.lax.broadcasted_iota(jnp.int32, logits.shape, 1)
    l = jnp.where(lane < E, logits, NEG)
    m1 = jnp.max(l, axis=1, keepdims=True)
    i1 = jnp.min(jnp.where(l == m1, lane, EPAD - 1), axis=1, keepdims=True)
    l2 = jnp.where(lane == i1, NEG, l)
    m2 = jnp.max(l2, axis=1, keepdims=True)
    i2 = jnp.min(jnp.where(l2 == m2, lane, EPAD - 1), axis=1, keepdims=True)
    oh1 = lane == i1
    oh2 = lane == i2
    mask = (oh1 | oh2).astype(jnp.float32)
    lt = lt_ref[...]
    run = jnp.zeros((1, EPAD), jnp.float32)
    parts = []
    for bk in range(N_TOK // BLK):
        m_b = mask[bk * BLK:(bk + 1) * BLK, :]
        c_b = jax.lax.dot_general(lt, m_b, (((1,), (0,)), ((), ())),
                                  preferred_element_type=jnp.float32)
        parts.append(c_b + run)
        run = run + c_b[BLK - 1:BLK, :]
    cum = jnp.concatenate(parts, axis=0)
    counts = run.astype(jnp.int32)
    padded = ((counts + (TM - 1)) >> TM_LOG2) << TM_LOG2
    lane_r = jax.lax.broadcasted_iota(jnp.int32, (1, EPAD), 1)
    offs = jnp.zeros((1, EPAD), jnp.int32)
    for e in range(E - 1):
        p_e = jnp.sum(jnp.where(lane_r == e, padded, 0))
        offs = offs + jnp.where(lane_r > e, p_e, 0)
    pos = offs.astype(jnp.float32) + cum - 1.0
    pos1_ref[...] = jnp.sum(jnp.where(oh1, pos, 0.0), axis=1,
                            keepdims=True).astype(jnp.int32)
    pos2_ref[...] = jnp.sum(jnp.where(oh2, pos, 0.0), axis=1,
                            keepdims=True).astype(jnp.int32)
    lane_row = jax.lax.broadcasted_iota(jnp.int32, (1, EPAD), 1)
    tl = lane_row * TM
    et = jnp.zeros((1, EPAD), jnp.int32)
    for e in range(E):
        off_e = jnp.sum(jnp.where(lane_row == e, offs, 0))
        et = et + (tl >= off_e).astype(jnp.int32)
    total = jnp.sum(jnp.where(lane_row < E, padded, 0))
    e_last = jnp.max(jnp.where(tl < total, et, 0))
    et_ref[...] = jnp.where(tl < total, et, -e_last)


def _dispatch_kernel(xf_hbm, pos1_hbm, pos2_hbm, xg_hbm,
                     idx1_v, idx2_v, rows_v, sem_in, sem_s1, sem_s2):
    wid = lax.axis_index("s") * NC + lax.axis_index("c")
    base = wid * TPW
    pltpu.sync_copy(pos1_hbm.at[wid], idx1_v)
    pltpu.sync_copy(pos2_hbm.at[wid], idx2_v)
    for j in range(NCH):
        pltpu.async_copy(xf_hbm.at[pl.ds(base + j * CH, CH)], rows_v,
                         sem_in).wait()
        cp1 = pltpu.async_copy(rows_v, xg_hbm.at[idx1_v.at[j]], sem_s1)
        cp2 = pltpu.async_copy(rows_v, xg_hbm.at[idx2_v.at[j]], sem_s2)
        cp1.wait()
        cp2.wait()


def _gemm_kernel(vt_ref, xg_ref, v1_ref, v2_ref, g1_ref, b1_ref, g2_ref,
                 b2_ref, o_ref, s1_ref, s2_ref):
    t = pl.program_id(0)
    w1 = v1_ref[0]
    w2 = v2_ref[0]
    valid = vt_ref[t] > 0
    first = t == 0
    changed = vt_ref[t] != vt_ref[jnp.maximum(t - 1, 0)]

    @pl.when(valid)
    def _():
        @pl.when(jnp.logical_or(first, changed))
        def _():
            w1f = w1.astype(jnp.float32)
            ss1 = jnp.sum(w1f * w1f, axis=0, keepdims=True)
            s1_ref[...] = g1_ref[0] * jax.lax.rsqrt(jnp.maximum(ss1, 1e-24))
            w2f = w2.astype(jnp.float32)
            ss2 = jnp.sum(w2f * w2f, axis=0, keepdims=True)
            s2_ref[...] = g2_ref[0] * jax.lax.rsqrt(jnp.maximum(ss2, 1e-24))

        x = xg_ref[...].astype(jnp.bfloat16)
        h = jax.lax.dot_general(x, w1, (((1,), (0,)), ((), ())),
                                preferred_element_type=jnp.float32)
        h = h * s1_ref[...] + b1_ref[0]
        h = h / (1.0 + jnp.exp(-h))
        hb = h.astype(jnp.bfloat16)
        y = jax.lax.dot_general(hb, w2, (((1,), (0,)), ((), ())),
                                preferred_element_type=jnp.float32)
        o_ref[...] = y * s2_ref[...] + b2_ref[0]


def _combine_kernel(yg_hbm, pos1_hbm, pos2_hbm, out_hbm,
                    idx1_v, idx2_v, r1_v, r2_v, sem_g1, sem_g2):
    wid = lax.axis_index("s") * NC + lax.axis_index("c")
    base = wid * TPW
    pltpu.sync_copy(pos1_hbm.at[wid], idx1_v)
    pltpu.sync_copy(pos2_hbm.at[wid], idx2_v)
    for j in range(NCH):
        pltpu.async_copy(yg_hbm.at[idx1_v.at[j]], r1_v, sem_g1).wait()
        pltpu.async_copy(yg_hbm.at[idx2_v.at[j]], r2_v, sem_g2).wait()

        def body(c, _):
            for r in range(CH):
                sl = pl.ds(c * 16, 16)
                r1_v[r, sl] = r1_v[r, sl] + r2_v[r, sl]
            return 0

        lax.fori_loop(0, D // 16, body, 0)
        pltpu.sync_copy(r1_v, out_hbm.at[pl.ds(base + j * CH, CH)])


def kernel(x, v1, g1, b1, v2, g2, b2, gv, gg, gb):
    b, s, d = x.shape
    xf = x.reshape(-1, d)
    n = jnp.maximum(jnp.linalg.norm(gv, axis=1, keepdims=True), 1e-12)
    wg = (gv / n) * gg[:, None]
    wg_pad = jnp.zeros((EPAD, D), jnp.float32).at[:E].set(wg)
    gb_pad = jnp.zeros((1, EPAD), jnp.float32).at[0, :E].set(gb)

    lt = jnp.tril(jnp.ones((BLK, BLK), jnp.float32))

    pos1, pos2, et = pl.pallas_call(
        _gate_kernel,
        out_shape=[jax.ShapeDtypeStruct((N_TOK, 1), jnp.int32),
                   jax.ShapeDtypeStruct((N_TOK, 1), jnp.int32),
                   jax.ShapeDtypeStruct((1, EPAD), jnp.int32)],
    )(xf, wg_pad, gb_pad, lt)

    pos1r = pos1.reshape(NW, NCH, CH)
    pos2r = pos2.reshape(NW, NCH, CH)
    et_flat = et.reshape(EPAD)[:NT]

    mesh = plsc.VectorSubcoreMesh(core_axis_name="c", subcore_axis_name="s")
    xg = pl.kernel(
        _dispatch_kernel,
        out_type=jax.ShapeDtypeStruct((P_ROWS, D), jnp.float32),
        mesh=mesh,
        scratch_types=[pltpu.VMEM((NCH, CH), jnp.int32),
                       pltpu.VMEM((NCH, CH), jnp.int32),
                       pltpu.VMEM((CH, D), jnp.float32),
                       pltpu.SemaphoreType.DMA,
                       pltpu.SemaphoreType.DMA,
                       pltpu.SemaphoreType.DMA],
    )(xf, pos1r, pos2r)

    v1t = jnp.swapaxes(v1.astype(jnp.bfloat16), 1, 2)
    v2t = jnp.swapaxes(v2.astype(jnp.bfloat16), 1, 2)

    yg = pl.pallas_call(
        _gemm_kernel,
        grid_spec=pltpu.PrefetchScalarGridSpec(
            num_scalar_prefetch=1,
            grid=(NT,),
            in_specs=[
                pl.BlockSpec((TM, D), lambda t, vt: (t, 0)),
                pl.BlockSpec((1, D, H), lambda t, vt: (jnp.abs(vt[t]) - 1, 0, 0)),
                pl.BlockSpec((1, H, D), lambda t, vt: (jnp.abs(vt[t]) - 1, 0, 0)),
                pl.BlockSpec((1, 1, H), lambda t, vt: (jnp.abs(vt[t]) - 1, 0, 0)),
                pl.BlockSpec((1, 1, H), lambda t, vt: (jnp.abs(vt[t]) - 1, 0, 0)),
                pl.BlockSpec((1, 1, D), lambda t, vt: (jnp.abs(vt[t]) - 1, 0, 0)),
                pl.BlockSpec((1, 1, D), lambda t, vt: (jnp.abs(vt[t]) - 1, 0, 0)),
            ],
            out_specs=pl.BlockSpec((TM, D), lambda t, vt: (t, 0)),
            scratch_shapes=[pltpu.VMEM((1, H), jnp.float32),
                            pltpu.VMEM((1, D), jnp.float32)],
        ),
        out_shape=jax.ShapeDtypeStruct((P_ROWS, D), jnp.float32),
        compiler_params=pltpu.CompilerParams(
            vmem_limit_bytes=100 * 1024 * 1024),
    )(et_flat, xg, v1t, v2t, g1.reshape(E, 1, H), b1.reshape(E, 1, H),
      g2.reshape(E, 1, D), b2.reshape(E, 1, D))

    out = pl.kernel(
        _combine_kernel,
        out_type=jax.ShapeDtypeStruct((N_TOK, D), jnp.float32),
        mesh=mesh,
        scratch_types=[pltpu.VMEM((NCH, CH), jnp.int32),
                       pltpu.VMEM((NCH, CH), jnp.int32),
                       pltpu.VMEM((CH, D), jnp.float32),
                       pltpu.VMEM((CH, D), jnp.float32),
                       pltpu.SemaphoreType.DMA,
                       pltpu.SemaphoreType.DMA],
    )(yg, pos1r, pos2r)

    return out.reshape(b, s, d)
